# baseline (speedup 1.0000x reference)
import jax
import jax.numpy as jnp
from jax import lax
from jax.experimental import pallas as pl
from jax.experimental.pallas import tpu as pltpu
from jax.experimental.pallas import tpu_sc as plsc

B = 16384
F = 26
FIELD = 38461
NC, NS, L = 2, 16, 16
NW = NC * NS
BPW = B // NW
E = BPW * F
VECS = E // L
CHUNK = 128
NCH = E // CHUNK


def _sc_body(x_hbm, tbl_hbm, bias_hbm, out_hbm, xv, idxv, rows, outv, biasv, sem):
    wid = lax.axis_index("s") * NC + lax.axis_index("c")
    base = wid * E
    pltpu.sync_copy(x_hbm.at[pl.ds(base, E)], xv)
    pltpu.sync_copy(bias_hbm, biasv)

    iota = lax.iota(jnp.int32, L)

    def build(i, carry):
        off = pl.multiple_of(i * L, L)
        pos = iota + off
        idxv[pl.ds(off, L)] = xv[pl.ds(off, L)] + (pos % F) * FIELD
        return carry

    lax.fori_loop(0, VECS, build, 0)

    def fire(j, carry):
        off = pl.multiple_of(j * CHUNK, CHUNK)
        pltpu.async_copy(
            tbl_hbm.at[idxv.at[pl.ds(off, CHUNK)]],
            rows.at[pl.ds(off, CHUNK)],
            sem,
        )
        return carry

    lax.fori_loop(0, NCH, fire, 0)
    pltpu.make_async_copy(tbl_hbm.at[pl.ds(0, E)], rows, sem).wait()

    bias16 = biasv[...]
    lanes = iota * F

    def reduce(c, carry):
        cbase = c * (L * F)
        acc = bias16
        for f in range(F):
            acc = acc + plsc.load_gather(rows, [lanes + (cbase + f)])
        outv[pl.ds(pl.multiple_of(c * L, L), L)] = acc
        return carry

    lax.fori_loop(0, BPW // L, reduce, 0)

    pltpu.sync_copy(outv, out_hbm.at[pl.ds(wid * BPW, BPW)])


def kernel(x, table, bias):
    xf = x.astype(jnp.int32).reshape(-1)
    tbl = table.reshape(-1)
    bias16 = jnp.broadcast_to(bias.astype(jnp.float32), (L,))
    mesh = plsc.VectorSubcoreMesh(
        core_axis_name="c", subcore_axis_name="s",
        num_cores=NC, num_subcores=NS,
    )
    out = pl.kernel(
        _sc_body,
        out_type=jax.ShapeDtypeStruct((B,), jnp.float32),
        mesh=mesh,
        compiler_params=pltpu.CompilerParams(needs_layout_passes=False),
        scratch_types=[
            pltpu.VMEM((E,), jnp.int32),
            pltpu.VMEM((E,), jnp.int32),
            pltpu.VMEM((E,), jnp.float32),
            pltpu.VMEM((BPW,), jnp.float32),
            pltpu.VMEM((L,), jnp.float32),
            pltpu.SemaphoreType.DMA,
        ],
    )(xf, tbl, bias16)
    return out.reshape(B, 1)


# speedup vs baseline: 1.9776x; 1.9776x over previous
import jax
import jax.numpy as jnp
from jax import lax
from jax.experimental import pallas as pl
from jax.experimental.pallas import tpu as pltpu
from jax.experimental.pallas import tpu_sc as plsc

B = 16384
F = 26
FIELD = 38461
NC, NS, L = 2, 16, 16
NW = NC * NS
BPW = B // NW
E = BPW * F
VPF = BPW // L
CHUNK = 128
NCH = E // CHUNK


def _sc_body(xt_hbm, tbl_hbm, bias_hbm, out_hbm, xv, idxv, rows, outv, biasv, sem):
    wid = lax.axis_index("s") * NC + lax.axis_index("c")
    bbase = wid * BPW

    for f in range(F):
        pltpu.async_copy(
            xt_hbm.at[f, pl.ds(bbase, BPW)],
            xv.at[pl.ds(f * BPW, BPW)],
            sem,
        )
    pltpu.sync_copy(bias_hbm, biasv)
    pltpu.make_async_copy(xt_hbm.at[0, pl.ds(0, E)], xv, sem).wait()

    def build_field(f, carry):
        fbase = pl.multiple_of(f * BPW, BPW)
        off_vec = jnp.full((L,), f * FIELD, dtype=jnp.int32)

        def build_vec(v, carry2):
            off = pl.multiple_of(fbase + v * L, L)
            idxv[pl.ds(off, L)] = xv[pl.ds(off, L)] + off_vec
            return carry2

        lax.fori_loop(0, VPF, build_vec, 0)
        return carry

    lax.fori_loop(0, F, build_field, 0)

    def fire(j, carry):
        off = pl.multiple_of(j * CHUNK, CHUNK)
        pltpu.async_copy(
            tbl_hbm.at[idxv.at[pl.ds(off, CHUNK)]],
            rows.at[pl.ds(off, CHUNK)],
            sem,
        )
        return carry

    lax.fori_loop(0, NCH, fire, 0)
    pltpu.make_async_copy(tbl_hbm.at[pl.ds(0, E)], rows, sem).wait()

    bias16 = biasv[...]

    def reduce(c, carry):
        cbase = pl.multiple_of(c * L, L)
        acc = bias16
        for f in range(F):
            acc = acc + rows[pl.ds(f * BPW + cbase, L)]
        outv[pl.ds(cbase, L)] = acc
        return carry

    lax.fori_loop(0, VPF, reduce, 0)

    pltpu.sync_copy(outv, out_hbm.at[pl.ds(wid * BPW, BPW)])


def kernel(x, table, bias):
    xt = x.astype(jnp.int32).T
    tbl = jnp.pad(table, ((0, 462), (0, 0))).reshape(-1)
    bias16 = jnp.broadcast_to(bias.astype(jnp.float32), (L,))
    mesh = plsc.VectorSubcoreMesh(
        core_axis_name="c", subcore_axis_name="s",
        num_cores=NC, num_subcores=NS,
    )
    out = pl.kernel(
        _sc_body,
        out_type=jax.ShapeDtypeStruct((B,), jnp.float32),
        mesh=mesh,
        compiler_params=pltpu.CompilerParams(needs_layout_passes=False),
        scratch_types=[
            pltpu.VMEM((E,), jnp.int32),
            pltpu.VMEM((E,), jnp.int32),
            pltpu.VMEM((E,), jnp.float32),
            pltpu.VMEM((BPW,), jnp.float32),
            pltpu.VMEM((L,), jnp.float32),
            pltpu.SemaphoreType.DMA,
        ],
    )(xt, tbl, bias16)
    return out.reshape(B, 1)


# speedup vs baseline: 2.1538x; 1.0891x over previous
import jax
import jax.numpy as jnp
from jax import lax
from jax.experimental import pallas as pl
from jax.experimental.pallas import tpu as pltpu
from jax.experimental.pallas import tpu_sc as plsc

B = 16384
F = 26
FIELD = 38461
NC, NS, L = 2, 16, 16
NW = NC * NS
BPW = B // NW
E = BPW * F
VPF = BPW // L
CHUNK = 128
NCH = E // CHUNK


def _sc_body(xt_hbm, tbl_hbm, bias_hbm, out_hbm, xv, idxv, rows, outv, biasv, sem):
    wid = lax.axis_index("s") * NC + lax.axis_index("c")
    bbase = wid * BPW

    for f in range(F):
        pltpu.async_copy(
            xt_hbm.at[f, pl.ds(bbase, BPW)],
            xv.at[pl.ds(f * BPW, BPW)],
            sem,
        )
    pltpu.sync_copy(bias_hbm, biasv)
    pltpu.make_async_copy(xt_hbm.at[0, pl.ds(0, E)], xv, sem).wait()

    UNROLL = 4
    CPF = BPW // CHUNK

    def build_field(f, carry):
        fbase = pl.multiple_of(f * BPW, BPW)
        off_vec = jnp.full((L,), f * FIELD, dtype=jnp.int32)

        def build_vec(v, carry2):
            for u in range(UNROLL):
                off = pl.multiple_of(fbase + (v * UNROLL + u) * L, L)
                idxv[pl.ds(off, L)] = xv[pl.ds(off, L)] + off_vec
            return carry2

        lax.fori_loop(0, VPF // UNROLL, build_vec, 0)
        for j in range(CPF):
            off = pl.multiple_of(fbase + j * CHUNK, CHUNK)
            pltpu.async_copy(
                tbl_hbm.at[idxv.at[pl.ds(off, CHUNK)]],
                rows.at[pl.ds(off, CHUNK)],
                sem,
            )
        return carry

    lax.fori_loop(0, F, build_field, 0)
    pltpu.make_async_copy(tbl_hbm.at[pl.ds(0, E)], rows, sem).wait()

    bias16 = biasv[...]

    def reduce(c, carry):
        cbase = pl.multiple_of(c * L, L)
        acc = bias16
        for f in range(F):
            acc = acc + rows[pl.ds(f * BPW + cbase, L)]
        outv[pl.ds(cbase, L)] = acc
        return carry

    lax.fori_loop(0, VPF, reduce, 0)

    pltpu.sync_copy(outv, out_hbm.at[pl.ds(wid * BPW, BPW)])


def kernel(x, table, bias):
    xt = x.astype(jnp.int32).T
    tbl = jnp.pad(table, ((0, 462), (0, 0))).reshape(-1)
    bias16 = jnp.broadcast_to(bias.astype(jnp.float32), (L,))
    mesh = plsc.VectorSubcoreMesh(
        core_axis_name="c", subcore_axis_name="s",
        num_cores=NC, num_subcores=NS,
    )
    out = pl.kernel(
        _sc_body,
        out_type=jax.ShapeDtypeStruct((B,), jnp.float32),
        mesh=mesh,
        compiler_params=pltpu.CompilerParams(needs_layout_passes=False),
        scratch_types=[
            pltpu.VMEM((E,), jnp.int32),
            pltpu.VMEM((E,), jnp.int32),
            pltpu.VMEM((E,), jnp.float32),
            pltpu.VMEM((BPW,), jnp.float32),
            pltpu.VMEM((L,), jnp.float32),
            pltpu.SemaphoreType.DMA,
        ],
    )(xt, tbl, bias16)
    return out.reshape(B, 1)


# speedup vs baseline: 2.1599x; 1.0028x over previous
import jax
import jax.numpy as jnp
from jax import lax
from jax.experimental import pallas as pl
from jax.experimental.pallas import tpu as pltpu
from jax.experimental.pallas import tpu_sc as plsc

B = 16384
F = 26
FIELD = 38461
NC, NS, L = 2, 16, 16
NW = NC * NS
BPW = B // NW
E = BPW * F
VPF = BPW // L
CHUNK = 512
NCH = E // CHUNK


def _sc_body(xt_hbm, tbl_hbm, bias_hbm, out_hbm, xv, idxv, rows, outv, biasv, sem):
    wid = lax.axis_index("s") * NC + lax.axis_index("c")
    bbase = wid * BPW

    for f in range(F):
        pltpu.async_copy(
            xt_hbm.at[f, pl.ds(bbase, BPW)],
            xv.at[pl.ds(f * BPW, BPW)],
            sem,
        )
    pltpu.sync_copy(bias_hbm, biasv)
    pltpu.make_async_copy(xt_hbm.at[0, pl.ds(0, E)], xv, sem).wait()

    UNROLL = 4
    CPF = BPW // CHUNK

    def build_field(f, carry):
        fbase = pl.multiple_of(f * BPW, BPW)
        off_vec = jnp.full((L,), f * FIELD, dtype=jnp.int32)

        def build_vec(v, carry2):
            for u in range(UNROLL):
                off = pl.multiple_of(fbase + (v * UNROLL + u) * L, L)
                idxv[pl.ds(off, L)] = xv[pl.ds(off, L)] + off_vec
            return carry2

        lax.fori_loop(0, VPF // UNROLL, build_vec, 0)
        for j in range(CPF):
            off = pl.multiple_of(fbase + j * CHUNK, CHUNK)
            pltpu.async_copy(
                tbl_hbm.at[idxv.at[pl.ds(off, CHUNK)]],
                rows.at[pl.ds(off, CHUNK)],
                sem,
            )
        return carry

    lax.fori_loop(0, F, build_field, 0)
    pltpu.make_async_copy(tbl_hbm.at[pl.ds(0, E)], rows, sem).wait()

    bias16 = biasv[...]

    def reduce(c, carry):
        cbase = pl.multiple_of(c * L, L)
        acc = bias16
        for f in range(F):
            acc = acc + rows[pl.ds(f * BPW + cbase, L)]
        outv[pl.ds(cbase, L)] = acc
        return carry

    lax.fori_loop(0, VPF, reduce, 0)

    pltpu.sync_copy(outv, out_hbm.at[pl.ds(wid * BPW, BPW)])


def kernel(x, table, bias):
    xt = x.astype(jnp.int32).T
    tbl = jnp.pad(table, ((0, 462), (0, 0))).reshape(-1)
    bias16 = jnp.broadcast_to(bias.astype(jnp.float32), (L,))
    mesh = plsc.VectorSubcoreMesh(
        core_axis_name="c", subcore_axis_name="s",
        num_cores=NC, num_subcores=NS,
    )
    out = pl.kernel(
        _sc_body,
        out_type=jax.ShapeDtypeStruct((B,), jnp.float32),
        mesh=mesh,
        compiler_params=pltpu.CompilerParams(needs_layout_passes=False),
        scratch_types=[
            pltpu.VMEM((E,), jnp.int32),
            pltpu.VMEM((E,), jnp.int32),
            pltpu.VMEM((E,), jnp.float32),
            pltpu.VMEM((BPW,), jnp.float32),
            pltpu.VMEM((L,), jnp.float32),
            pltpu.SemaphoreType.DMA,
        ],
    )(xt, tbl, bias16)
    return out.reshape(B, 1)


# speedup vs baseline: 2.1610x; 1.0005x over previous
import jax
import jax.numpy as jnp
from jax import lax
from jax.experimental import pallas as pl
from jax.experimental.pallas import tpu as pltpu
from jax.experimental.pallas import tpu_sc as plsc

B = 16384
F = 26
FIELD = 38461
NC, NS, L = 2, 16, 16
NW = NC * NS
BPW = B // NW
E = BPW * F
VPF = BPW // L
CHUNK = 512
NCH = E // CHUNK


def _sc_body(xt_hbm, tbl_hbm, bias_hbm, out_hbm, xv, idxv, rows, outv, biasv, sem):
    wid = lax.axis_index("s") * NC + lax.axis_index("c")
    bbase = wid * BPW

    with jax.named_scope("x_stage"):
        for f in range(F):
            pltpu.async_copy(
                xt_hbm.at[f, pl.ds(bbase, BPW)],
                xv.at[pl.ds(f * BPW, BPW)],
                sem,
            )
        pltpu.sync_copy(bias_hbm, biasv)
        pltpu.make_async_copy(xt_hbm.at[0, pl.ds(0, E)], xv, sem).wait()

    UNROLL = 4
    CPF = BPW // CHUNK

    def build_field(f, carry):
        fbase = pl.multiple_of(f * BPW, BPW)
        off_vec = jnp.full((L,), f * FIELD, dtype=jnp.int32)

        def build_vec(v, carry2):
            for u in range(UNROLL):
                off = pl.multiple_of(fbase + (v * UNROLL + u) * L, L)
                idxv[pl.ds(off, L)] = xv[pl.ds(off, L)] + off_vec
            return carry2

        lax.fori_loop(0, VPF // UNROLL, build_vec, 0)
        for j in range(CPF):
            off = pl.multiple_of(fbase + j * CHUNK, CHUNK)
            pltpu.async_copy(
                tbl_hbm.at[idxv.at[pl.ds(off, CHUNK)]],
                rows.at[pl.ds(off, CHUNK)],
                sem,
            )
        return carry

    with jax.named_scope("build_fire"):
        lax.fori_loop(0, F, build_field, 0)
    with jax.named_scope("drain"):
        pltpu.make_async_copy(tbl_hbm.at[pl.ds(0, E)], rows, sem).wait()

    bias16 = biasv[...]

    def reduce(c, carry):
        cbase = pl.multiple_of(c * L, L)
        acc = bias16
        for f in range(F):
            acc = acc + rows[pl.ds(f * BPW + cbase, L)]
        outv[pl.ds(cbase, L)] = acc
        return carry

    with jax.named_scope("reduce"):
        lax.fori_loop(0, VPF, reduce, 0)

    with jax.named_scope("writeback"):
        pltpu.sync_copy(outv, out_hbm.at[pl.ds(wid * BPW, BPW)])


def kernel(x, table, bias):
    xt = x.astype(jnp.int32).T
    tbl = jnp.pad(table, ((0, 462), (0, 0))).reshape(-1)
    bias16 = jnp.broadcast_to(bias.astype(jnp.float32), (L,))
    mesh = plsc.VectorSubcoreMesh(
        core_axis_name="c", subcore_axis_name="s",
        num_cores=NC, num_subcores=NS,
    )
    out = pl.kernel(
        _sc_body,
        out_type=jax.ShapeDtypeStruct((B,), jnp.float32),
        mesh=mesh,
        compiler_params=pltpu.CompilerParams(needs_layout_passes=False),
        scratch_types=[
            pltpu.VMEM((E,), jnp.int32),
            pltpu.VMEM((E,), jnp.int32),
            pltpu.VMEM((E,), jnp.float32),
            pltpu.VMEM((BPW,), jnp.float32),
            pltpu.VMEM((L,), jnp.float32),
            pltpu.SemaphoreType.DMA,
        ],
    )(xt, tbl, bias16)
    return out.reshape(B, 1)
